# baseline (speedup 1.0000x reference)
import functools

import jax
import jax.numpy as jnp
from jax import lax
from jax.experimental import pallas as pl
from jax.experimental.pallas import tpu as pltpu
from jax.experimental.pallas import tpu_sc as plsc

N = 10000
E = 640000
D = 128
H = 64
G = 64

CHUNK = 128
NUM_CHUNKS = E // CHUNK
NC = 2
NS = 16
NW = NC * NS
BLK = 4
NUM_BLOCKS = NUM_CHUNKS // BLK
BLOCKS_MAIN = NUM_BLOCKS // NW
UNROLL = 12
OUTER = BLOCKS_MAIN * BLK // UNROLL
PAD = 10240
SUB_ROWS = PAD // NS


def _make_seg_sum(width):
  mesh = plsc.VectorSubcoreMesh(core_axis_name="c", subcore_axis_name="s")

  @functools.partial(
      pl.kernel,
      out_type=jax.ShapeDtypeStruct((2 * PAD, width), jnp.float32),
      mesh=mesh,
      compiler_params=pltpu.CompilerParams(use_tc_tiling_on_sc=False),
      scratch_types=[
          pltpu.VMEM((3, 2, BLK, CHUNK), jnp.int32),
          pltpu.VMEM((4, CHUNK, width), jnp.float32),
          pltpu.VMEM_SHARED((PAD, width), jnp.float32),
          pltpu.VMEM_SHARED((PAD, width), jnp.float32),
          pltpu.SemaphoreType.DMA,
          pltpu.SemaphoreType.DMA,
          pltpu.SemaphoreType.DMA,
          pltpu.SemaphoreType.DMA,
          pltpu.SemaphoreType.DMA,
          pltpu.SemaphoreType.DMA,
          pltpu.SemaphoreType.DMA,
          pltpu.SemaphoreType.DMA,
          pltpu.SemaphoreType.DMA,
          pltpu.SemaphoreType.DMA,
          pltpu.SemaphoreType.DMA,
      ],
  )
  def seg_sum(feat_hbm, ei_hbm, zeros_hbm, out_hbm,
              eblk, rows, acc_sh, feat_sh, bsem0, bsem1, bsem2,
              gsem0, gsem1, gsem2, gsem3, ssem0, ssem1, ssem2, ssem3):
    bsem = [bsem0, bsem1, bsem2]
    gsem = [gsem0, gsem1, gsem2, gsem3]
    ssem = [ssem0, ssem1, ssem2, ssem3]
    c = lax.axis_index("c")
    s = lax.axis_index("s")
    wid = s * NC + c

    def blk_src(g):
      return ei_hbm.at[:, pl.ds((g * NW + wid) * BLK, BLK), :]

    def gather_desc(B, p):
      return pltpu.make_async_copy(feat_sh.at[eblk.at[B, 0, p]],
                                   rows.at[p], gsem[p])

    def scatter_desc(B, p):
      return pltpu.make_async_copy(rows.at[p], acc_sh.at[eblk.at[B, 1, p]],
                                   ssem[p])

    def scatter_start(B, p):
      pltpu.async_copy(rows.at[p], acc_sh.at[eblk.at[B, 1, p]], ssem[p],
                       add=True)

    pltpu.sync_copy(zeros_hbm,
                    acc_sh.at[pl.ds(s * SUB_ROWS, SUB_ROWS)])

    @pl.when(s < NS - 1)
    def _():
      pltpu.sync_copy(feat_hbm.at[pl.ds(s * SUB_ROWS, SUB_ROWS)],
                      feat_sh.at[pl.ds(s * SUB_ROWS, SUB_ROWS)])

    @pl.when(s == NS - 1)
    def _():
      pltpu.sync_copy(feat_hbm.at[pl.ds((NS - 1) * SUB_ROWS, N - (NS - 1) * SUB_ROWS)],
                      feat_sh.at[pl.ds((NS - 1) * SUB_ROWS, N - (NS - 1) * SUB_ROWS)])
    pltpu.async_copy(blk_src(0), eblk.at[0], bsem[0])
    pltpu.async_copy(blk_src(1), eblk.at[1], bsem[1])
    plsc.subcore_barrier()

    def outer(t, carry):
      for u in range(UNROLL):
        k = t * UNROLL + u
        g = t * 3 + u // BLK
        B = (u // BLK) % 3
        p = u % BLK
        if p == 0:
          pltpu.make_async_copy(blk_src(g), eblk.at[B], bsem[B]).wait()

        @pl.when(k >= 4)
        def _():
          scatter_desc(((u + 8) // BLK) % 3, p).wait()

        gather_desc(B, p).start()
        pB, pp = ((u + 9) // BLK) % 3, (u + 1) % 4

        @pl.when(k >= 3)
        def _():
          gather_desc(pB, pp).wait()
          scatter_start(pB, pp)

        if p == 3:
          @pl.when(g + 2 < BLOCKS_MAIN)
          def _():
            nB = (u // BLK + 2) % 3
            pltpu.async_copy(blk_src(g + 2), eblk.at[nB], bsem[nB])
      return carry

    lax.fori_loop(0, OUTER, outer, 0)
    gather_desc(2, 1).wait()
    scatter_start(2, 1)
    gather_desc(2, 2).wait()
    scatter_start(2, 2)
    gather_desc(2, 3).wait()
    scatter_start(2, 3)
    scatter_desc(2, 0).wait()
    scatter_desc(2, 1).wait()
    scatter_desc(2, 2).wait()
    scatter_desc(2, 3).wait()

    @pl.when(wid < NUM_BLOCKS - BLOCKS_MAIN * NW)
    def _():
      pltpu.sync_copy(blk_src(BLOCKS_MAIN), eblk.at[0])
      for p in range(BLK):
        gather_desc(0, p).start()
        gather_desc(0, p).wait()
        pltpu.sync_copy(rows.at[p], acc_sh.at[eblk.at[0, 1, p]], add=True)

    plsc.subcore_barrier()
    pltpu.sync_copy(acc_sh.at[pl.ds(s * SUB_ROWS, SUB_ROWS)],
                    out_hbm.at[pl.ds(c * PAD + s * SUB_ROWS, SUB_ROWS)])

  return seg_sum


_seg_sum_64 = _make_seg_sum(H)


N2 = N // 2
PAD2 = PAD // 2


def _dup(v):
  return jnp.concatenate([v, v], axis=0)


def _bdiag(w):
  z = jnp.zeros((H, H), jnp.float32)
  return jnp.concatenate(
      [jnp.concatenate([w, z], axis=1), jnp.concatenate([z, w], axis=1)],
      axis=0)


def _bn(a, gamma, beta, eps=1e-5):
  ones = jnp.ones((1, N2), jnp.float32)
  aa = jnp.concatenate([a, a * a], axis=1)
  s = lax.dot_general(ones, aa, (((1,), (0,)), ((), ())),
                      preferred_element_type=jnp.float32)
  m = (s[:, :H] + s[:, H:2 * H]) * (1.0 / N)
  q = (s[:, 2 * H:3 * H] + s[:, 3 * H:]) * (1.0 / N)
  v = jnp.maximum(q - m * m, 0.0)
  md = jnp.concatenate([m, m], axis=1)
  vd = jnp.concatenate([v, v], axis=1)
  return _dup(gamma) * (a - md) / jnp.sqrt(vd + eps) + _dup(beta)


def _mlp(h, p):
  a = jnp.dot(h, _bdiag(p["W1"][...]), preferred_element_type=jnp.float32)
  a = a + _dup(p["b1"][...])
  a = jnp.maximum(_bn(a, p["g1"][...], p["be1"][...]), 0.0)
  return _mlp_tail(a, p)


def _mlp_tail(a, p):
  a = jnp.dot(a, _bdiag(p["W2"][...]), preferred_element_type=jnp.float32)
  a = a + _dup(p["b2"][...])
  return jnp.maximum(_bn(a, p["g2"][...], p["be2"][...]), 0.0)


def _premul_body(x_ref, w_ref, o_ref):
  w = w_ref[...]
  z = jnp.zeros((D, H), jnp.float32)
  wb = jnp.concatenate(
      [jnp.concatenate([w, z], axis=1), jnp.concatenate([z, w], axis=1)],
      axis=0)
  o_ref[...] = jnp.dot(x_ref[...], wb, preferred_element_type=jnp.float32)


def _psum(p_ref):
  return p_ref[pl.ds(0, N2), :] + p_ref[pl.ds(PAD2, N2), :]


def _gin1_body(y_ref, p_ref, pr, o_ref):
  a = y_ref[...] + _psum(p_ref) + _dup(pr["b1"][...])
  a = jnp.maximum(_bn(a, pr["g1"][...], pr["be1"][...]), 0.0)
  o_ref[...] = _mlp_tail(a, pr)


def _gin_body(x_ref, p_ref, pr, o_ref):
  x = x_ref[...]
  o_ref[...] = x + _mlp(x + _psum(p_ref), pr)


def _head(g, p):
  t = jnp.maximum(
      jnp.dot(g, p["W1"][...], preferred_element_type=jnp.float32) + p["b1"][...], 0.0)
  z = jnp.dot(t, p["W2"][...], preferred_element_type=jnp.float32) + p["b2"][...]
  n = jnp.sqrt(jnp.sum(z * z, axis=1, keepdims=True))
  return z / jnp.maximum(n, 1e-12)


def _final_body(x_ref, p_ref, conv3_refs, batch_ref, hc_refs, hs_refs, hg_refs,
                ht_refs, oc_ref, os_ref, og_ref, ot_ref):
  x = x_ref[...]
  h3 = x + _mlp(x + _psum(p_ref), conv3_refs)
  gid = lax.broadcasted_iota(jnp.int32, (N2, G), 1)
  b2 = batch_ref[...]
  oh_e = (b2[:, 0:1] == gid).astype(jnp.float32)
  oh_o = (b2[:, 1:2] == gid).astype(jnp.float32)
  dn = (((0,), (0,)), ((), ()))
  g = (lax.dot_general(oh_e, h3[:, :H], dn, preferred_element_type=jnp.float32)
       + lax.dot_general(oh_o, h3[:, H:], dn,
                         preferred_element_type=jnp.float32))
  oc_ref[...] = _head(g, hc_refs)
  os_ref[...] = _head(g, hs_refs)
  og_ref[...] = _head(g, hg_refs)
  ot_ref[...] = _head(g, ht_refs)


def _nh(shape):
  return jax.ShapeDtypeStruct(shape, jnp.float32)


def pair2(p):
  return p.reshape(2 * PAD2, 2 * H)


def kernel(x, edge_index, batch, params):
  ei = edge_index.reshape(2, NUM_CHUNKS, CHUNK)
  zeros64 = jnp.zeros((SUB_ROWS, H), jnp.float32)

  def pair(a):
    return a.reshape(N2, 2 * H)

  def unpair(a):
    return a.reshape(N, H)

  y = pl.pallas_call(_premul_body, out_shape=_nh((N2, 2 * H)))(
      x.reshape(N2, 2 * D), params["conv1"]["W1"])
  p1 = pair2(_seg_sum_64(unpair(y), ei, zeros64))
  h1 = pl.pallas_call(_gin1_body, out_shape=_nh((N2, 2 * H)))(
      y, p1, params["conv1"])
  p2 = pair2(_seg_sum_64(unpair(h1), ei, zeros64))
  h2 = pl.pallas_call(_gin_body, out_shape=_nh((N2, 2 * H)))(
      h1, p2, params["conv2"])
  p3 = pair2(_seg_sum_64(unpair(h2), ei, zeros64))

  outs = pl.pallas_call(
      _final_body,
      out_shape=(_nh((G, 16)), _nh((G, 16)), _nh((G, 8)), _nh((G, 32))),
  )(h2, p3, params["conv3"], batch.reshape(N2, 2), params["head_color"],
    params["head_size"], params["head_ground"], params["head_struct"])
  return outs


# speedup vs baseline: 1.4299x; 1.4299x over previous
import functools

import jax
import jax.numpy as jnp
from jax import lax
from jax.experimental import pallas as pl
from jax.experimental.pallas import tpu as pltpu
from jax.experimental.pallas import tpu_sc as plsc

N = 10000
E = 640000
D = 128
H = 64
G = 64

CHUNK = 128
NUM_CHUNKS = E // CHUNK
NC = 2
NS = 16
NW = NC * NS
BLK = 4
NUM_BLOCKS = NUM_CHUNKS // BLK
BLOCKS_MAIN = NUM_BLOCKS // NW
UNROLL = 12
OUTER = BLOCKS_MAIN * BLK // UNROLL
PAD = 10240
SUB_ROWS = PAD // NS


def _make_seg_sum(width):
  mesh = plsc.VectorSubcoreMesh(core_axis_name="c", subcore_axis_name="s")

  @functools.partial(
      pl.kernel,
      out_type=jax.ShapeDtypeStruct((2 * PAD, width), jnp.float32),
      mesh=mesh,
      compiler_params=pltpu.CompilerParams(use_tc_tiling_on_sc=False),
      scratch_types=[
          pltpu.VMEM((3, 2, BLK, CHUNK), jnp.int32),
          pltpu.VMEM((4, CHUNK, width), jnp.float32),
          pltpu.VMEM_SHARED((PAD, width), jnp.float32),
          pltpu.SemaphoreType.DMA,
          pltpu.SemaphoreType.DMA,
          pltpu.SemaphoreType.DMA,
          pltpu.SemaphoreType.DMA,
          pltpu.SemaphoreType.DMA,
          pltpu.SemaphoreType.DMA,
          pltpu.SemaphoreType.DMA,
          pltpu.SemaphoreType.DMA,
          pltpu.SemaphoreType.DMA,
          pltpu.SemaphoreType.DMA,
          pltpu.SemaphoreType.DMA,
      ],
  )
  def seg_sum(feat_hbm, ei_hbm, zeros_hbm, out_hbm,
              eblk, rows, acc_sh, bsem0, bsem1, bsem2,
              gsem0, gsem1, gsem2, gsem3, ssem0, ssem1, ssem2, ssem3):
    bsem = [bsem0, bsem1, bsem2]
    gsem = [gsem0, gsem1, gsem2, gsem3]
    ssem = [ssem0, ssem1, ssem2, ssem3]
    c = lax.axis_index("c")
    s = lax.axis_index("s")
    wid = s * NC + c

    def blk_src(g):
      return ei_hbm.at[:, pl.ds((g * NW + wid) * BLK, BLK), :]

    def gather_desc(B, p):
      return pltpu.make_async_copy(feat_hbm.at[eblk.at[B, 0, p]],
                                   rows.at[p], gsem[p])

    def scatter_desc(B, p):
      return pltpu.make_async_copy(rows.at[p], acc_sh.at[eblk.at[B, 1, p]],
                                   ssem[p])

    def scatter_start(B, p):
      pltpu.async_copy(rows.at[p], acc_sh.at[eblk.at[B, 1, p]], ssem[p],
                       add=True)

    pltpu.sync_copy(zeros_hbm,
                    acc_sh.at[pl.ds(s * SUB_ROWS, SUB_ROWS)])
    pltpu.async_copy(blk_src(0), eblk.at[0], bsem[0])
    pltpu.async_copy(blk_src(1), eblk.at[1], bsem[1])
    plsc.subcore_barrier()

    def outer(t, carry):
      for u in range(UNROLL):
        k = t * UNROLL + u
        g = t * 3 + u // BLK
        B = (u // BLK) % 3
        p = u % BLK
        if p == 0:
          pltpu.make_async_copy(blk_src(g), eblk.at[B], bsem[B]).wait()

        @pl.when(k >= 4)
        def _():
          scatter_desc(((u + 8) // BLK) % 3, p).wait()

        gather_desc(B, p).start()
        pB, pp = ((u + 9) // BLK) % 3, (u + 1) % 4

        @pl.when(k >= 3)
        def _():
          gather_desc(pB, pp).wait()
          scatter_start(pB, pp)

        if p == 3:
          @pl.when(g + 2 < BLOCKS_MAIN)
          def _():
            nB = (u // BLK + 2) % 3
            pltpu.async_copy(blk_src(g + 2), eblk.at[nB], bsem[nB])
      return carry

    lax.fori_loop(0, OUTER, outer, 0)
    gather_desc(2, 1).wait()
    scatter_start(2, 1)
    gather_desc(2, 2).wait()
    scatter_start(2, 2)
    gather_desc(2, 3).wait()
    scatter_start(2, 3)
    scatter_desc(2, 0).wait()
    scatter_desc(2, 1).wait()
    scatter_desc(2, 2).wait()
    scatter_desc(2, 3).wait()

    @pl.when(wid < NUM_BLOCKS - BLOCKS_MAIN * NW)
    def _():
      pltpu.sync_copy(blk_src(BLOCKS_MAIN), eblk.at[0])
      for p in range(BLK):
        gather_desc(0, p).start()
        gather_desc(0, p).wait()
        pltpu.sync_copy(rows.at[p], acc_sh.at[eblk.at[0, 1, p]], add=True)

    plsc.subcore_barrier()
    pltpu.sync_copy(acc_sh.at[pl.ds(s * SUB_ROWS, SUB_ROWS)],
                    out_hbm.at[pl.ds(c * PAD + s * SUB_ROWS, SUB_ROWS)])

  return seg_sum


_seg_sum_64 = _make_seg_sum(H)


N2 = N // 2
PAD2 = PAD // 2


def _dup(v):
  return jnp.concatenate([v, v], axis=0)


def _bdiag(w):
  z = jnp.zeros((H, H), jnp.float32)
  return jnp.concatenate(
      [jnp.concatenate([w, z], axis=1), jnp.concatenate([z, w], axis=1)],
      axis=0)


def _bn(a, gamma, beta, eps=1e-5):
  ones = jnp.ones((1, N2), jnp.float32)
  aa = jnp.concatenate([a, a * a], axis=1)
  s = lax.dot_general(ones, aa, (((1,), (0,)), ((), ())),
                      preferred_element_type=jnp.float32)
  m = (s[:, :H] + s[:, H:2 * H]) * (1.0 / N)
  q = (s[:, 2 * H:3 * H] + s[:, 3 * H:]) * (1.0 / N)
  v = jnp.maximum(q - m * m, 0.0)
  md = jnp.concatenate([m, m], axis=1)
  vd = jnp.concatenate([v, v], axis=1)
  return _dup(gamma) * (a - md) / jnp.sqrt(vd + eps) + _dup(beta)


def _mlp(h, p):
  a = jnp.dot(h, _bdiag(p["W1"][...]), preferred_element_type=jnp.float32)
  a = a + _dup(p["b1"][...])
  a = jnp.maximum(_bn(a, p["g1"][...], p["be1"][...]), 0.0)
  return _mlp_tail(a, p)


def _mlp_tail(a, p):
  a = jnp.dot(a, _bdiag(p["W2"][...]), preferred_element_type=jnp.float32)
  a = a + _dup(p["b2"][...])
  return jnp.maximum(_bn(a, p["g2"][...], p["be2"][...]), 0.0)


def _premul_body(x_ref, w_ref, o_ref):
  w = w_ref[...]
  z = jnp.zeros((D, H), jnp.float32)
  wb = jnp.concatenate(
      [jnp.concatenate([w, z], axis=1), jnp.concatenate([z, w], axis=1)],
      axis=0)
  o_ref[...] = jnp.dot(x_ref[...], wb, preferred_element_type=jnp.float32)


def _psum(p_ref):
  return p_ref[pl.ds(0, N2), :] + p_ref[pl.ds(PAD2, N2), :]


def _gin1_body(y_ref, p_ref, pr, o_ref):
  a = y_ref[...] + _psum(p_ref) + _dup(pr["b1"][...])
  a = jnp.maximum(_bn(a, pr["g1"][...], pr["be1"][...]), 0.0)
  o_ref[...] = _mlp_tail(a, pr)


def _gin_body(x_ref, p_ref, pr, o_ref):
  x = x_ref[...]
  o_ref[...] = x + _mlp(x + _psum(p_ref), pr)


def _head(g, p):
  t = jnp.maximum(
      jnp.dot(g, p["W1"][...], preferred_element_type=jnp.float32) + p["b1"][...], 0.0)
  z = jnp.dot(t, p["W2"][...], preferred_element_type=jnp.float32) + p["b2"][...]
  n = jnp.sqrt(jnp.sum(z * z, axis=1, keepdims=True))
  return z / jnp.maximum(n, 1e-12)


def _final_body(x_ref, p_ref, conv3_refs, batch_ref, hc_refs, hs_refs, hg_refs,
                ht_refs, oc_ref, os_ref, og_ref, ot_ref):
  x = x_ref[...]
  h3 = x + _mlp(x + _psum(p_ref), conv3_refs)
  gid = lax.broadcasted_iota(jnp.int32, (N2, G), 1)
  b2 = batch_ref[...]
  oh_e = (b2[:, 0:1] == gid).astype(jnp.float32)
  oh_o = (b2[:, 1:2] == gid).astype(jnp.float32)
  dn = (((0,), (0,)), ((), ()))
  g = (lax.dot_general(oh_e, h3[:, :H], dn, preferred_element_type=jnp.float32)
       + lax.dot_general(oh_o, h3[:, H:], dn,
                         preferred_element_type=jnp.float32))
  oc_ref[...] = _head(g, hc_refs)
  os_ref[...] = _head(g, hs_refs)
  og_ref[...] = _head(g, hg_refs)
  ot_ref[...] = _head(g, ht_refs)


def _nh(shape):
  return jax.ShapeDtypeStruct(shape, jnp.float32)


def pair2(p):
  return p.reshape(2 * PAD2, 2 * H)


def kernel(x, edge_index, batch, params):
  ei = edge_index.reshape(2, NUM_CHUNKS, CHUNK)
  zeros64 = jnp.zeros((SUB_ROWS, H), jnp.float32)

  def pair(a):
    return a.reshape(N2, 2 * H)

  def unpair(a):
    return a.reshape(N, H)

  y = pl.pallas_call(_premul_body, out_shape=_nh((N2, 2 * H)))(
      x.reshape(N2, 2 * D), params["conv1"]["W1"])
  p1 = pair2(_seg_sum_64(unpair(y), ei, zeros64))
  h1 = pl.pallas_call(_gin1_body, out_shape=_nh((N2, 2 * H)))(
      y, p1, params["conv1"])
  p2 = pair2(_seg_sum_64(unpair(h1), ei, zeros64))
  h2 = pl.pallas_call(_gin_body, out_shape=_nh((N2, 2 * H)))(
      h1, p2, params["conv2"])
  p3 = pair2(_seg_sum_64(unpair(h2), ei, zeros64))

  outs = pl.pallas_call(
      _final_body,
      out_shape=(_nh((G, 16)), _nh((G, 16)), _nh((G, 8)), _nh((G, 32))),
  )(h2, p3, params["conv3"], batch.reshape(N2, 2), params["head_color"],
    params["head_size"], params["head_ground"], params["head_struct"])
  return outs


# speedup vs baseline: 1.4577x; 1.0194x over previous
import functools

import jax
import jax.numpy as jnp
from jax import lax
from jax.experimental import pallas as pl
from jax.experimental.pallas import tpu as pltpu
from jax.experimental.pallas import tpu_sc as plsc

N = 10000
E = 640000
D = 128
H = 64
G = 64

CHUNK = 128
NUM_CHUNKS = E // CHUNK
NC = 2
NS = 16
NW = NC * NS
BLK = 4
NUM_BLOCKS = NUM_CHUNKS // BLK
BLOCKS_MAIN = NUM_BLOCKS // NW
UNROLL = 12
OUTER = BLOCKS_MAIN * BLK // UNROLL
PAD = 10240
SUB_ROWS = PAD // NS


def _make_seg_sum(width):
  mesh = plsc.VectorSubcoreMesh(core_axis_name="c", subcore_axis_name="s")

  @functools.partial(
      pl.kernel,
      out_type=jax.ShapeDtypeStruct((2 * PAD, width), jnp.float32),
      mesh=mesh,
      compiler_params=pltpu.CompilerParams(use_tc_tiling_on_sc=False),
      scratch_types=[
          pltpu.VMEM((3, 2, BLK, CHUNK), jnp.int32),
          pltpu.VMEM((4, CHUNK, width), jnp.float32),
          pltpu.VMEM_SHARED((PAD, width), jnp.float32),
          pltpu.SemaphoreType.DMA,
          pltpu.SemaphoreType.DMA,
          pltpu.SemaphoreType.DMA,
          pltpu.SemaphoreType.DMA,
          pltpu.SemaphoreType.DMA,
          pltpu.SemaphoreType.DMA,
          pltpu.SemaphoreType.DMA,
          pltpu.SemaphoreType.DMA,
          pltpu.SemaphoreType.DMA,
          pltpu.SemaphoreType.DMA,
          pltpu.SemaphoreType.DMA,
      ],
  )
  def seg_sum(feat_hbm, ei_hbm, zeros_hbm, out_hbm,
              eblk, rows, acc_sh, bsem0, bsem1, bsem2,
              gsem0, gsem1, gsem2, gsem3, ssem0, ssem1, ssem2, ssem3):
    bsem = [bsem0, bsem1, bsem2]
    gsem = [gsem0, gsem1, gsem2, gsem3]
    ssem = [ssem0, ssem1, ssem2, ssem3]
    c = lax.axis_index("c")
    s = lax.axis_index("s")
    wid = s * NC + c

    def blk_src(g):
      return ei_hbm.at[:, pl.ds((g * NW + wid) * BLK, BLK), :]

    def gather_desc(B, p):
      return pltpu.make_async_copy(feat_hbm.at[eblk.at[B, 0, p]],
                                   rows.at[p], gsem[p])

    def scatter_desc(B, p):
      return pltpu.make_async_copy(rows.at[p], acc_sh.at[eblk.at[B, 1, p]],
                                   ssem[p])

    def scatter_start(B, p):
      pltpu.async_copy(rows.at[p], acc_sh.at[eblk.at[B, 1, p]], ssem[p],
                       add=True)

    pltpu.sync_copy(zeros_hbm,
                    acc_sh.at[pl.ds(s * SUB_ROWS, SUB_ROWS)])
    pltpu.async_copy(blk_src(0), eblk.at[0], bsem[0])
    pltpu.async_copy(blk_src(1), eblk.at[1], bsem[1])
    plsc.subcore_barrier()

    def outer(t, carry):
      for u in range(UNROLL):
        k = t * UNROLL + u
        g = t * 3 + u // BLK
        B = (u // BLK) % 3
        p = u % BLK
        if p == 0:
          pltpu.make_async_copy(blk_src(g), eblk.at[B], bsem[B]).wait()

        @pl.when(k >= 4)
        def _():
          scatter_desc(((u + 8) // BLK) % 3, p).wait()

        gather_desc(B, p).start()
        pB, pp = ((u + 9) // BLK) % 3, (u + 1) % 4

        @pl.when(k >= 3)
        def _():
          gather_desc(pB, pp).wait()
          scatter_start(pB, pp)

        if p == 3:
          @pl.when(g + 2 < BLOCKS_MAIN)
          def _():
            nB = (u // BLK + 2) % 3
            pltpu.async_copy(blk_src(g + 2), eblk.at[nB], bsem[nB])
      return carry

    lax.fori_loop(0, OUTER, outer, 0)
    gather_desc(2, 1).wait()
    scatter_start(2, 1)
    gather_desc(2, 2).wait()
    scatter_start(2, 2)
    gather_desc(2, 3).wait()
    scatter_start(2, 3)
    scatter_desc(2, 0).wait()
    scatter_desc(2, 1).wait()
    scatter_desc(2, 2).wait()
    scatter_desc(2, 3).wait()

    @pl.when(wid < NUM_BLOCKS - BLOCKS_MAIN * NW)
    def _():
      pltpu.sync_copy(blk_src(BLOCKS_MAIN), eblk.at[0])
      for p in range(BLK):
        gather_desc(0, p).start()
      for p in range(BLK):
        gather_desc(0, p).wait()
        pltpu.sync_copy(rows.at[p], acc_sh.at[eblk.at[0, 1, p]], add=True)

    plsc.subcore_barrier()
    pltpu.sync_copy(acc_sh.at[pl.ds(s * SUB_ROWS, SUB_ROWS)],
                    out_hbm.at[pl.ds(c * PAD + s * SUB_ROWS, SUB_ROWS)])

  return seg_sum


_seg_sum_64 = _make_seg_sum(H)


N2 = N // 2
PAD2 = PAD // 2


def _dup(v):
  return jnp.concatenate([v, v], axis=0)


def _bdiag(w):
  z = jnp.zeros((H, H), jnp.float32)
  return jnp.concatenate(
      [jnp.concatenate([w, z], axis=1), jnp.concatenate([z, w], axis=1)],
      axis=0)


def _bn(a, gamma, beta, eps=1e-5):
  ones = jnp.ones((1, N2), jnp.float32)
  aa = jnp.concatenate([a, a * a], axis=1)
  s = lax.dot_general(ones, aa, (((1,), (0,)), ((), ())),
                      preferred_element_type=jnp.float32)
  m = (s[:, :H] + s[:, H:2 * H]) * (1.0 / N)
  q = (s[:, 2 * H:3 * H] + s[:, 3 * H:]) * (1.0 / N)
  v = jnp.maximum(q - m * m, 0.0)
  md = jnp.concatenate([m, m], axis=1)
  vd = jnp.concatenate([v, v], axis=1)
  return _dup(gamma) * (a - md) / jnp.sqrt(vd + eps) + _dup(beta)


def _mlp(h, p):
  a = jnp.dot(h, _bdiag(p["W1"][...]), preferred_element_type=jnp.float32)
  a = a + _dup(p["b1"][...])
  a = jnp.maximum(_bn(a, p["g1"][...], p["be1"][...]), 0.0)
  return _mlp_tail(a, p)


def _mlp_tail(a, p):
  a = jnp.dot(a, _bdiag(p["W2"][...]), preferred_element_type=jnp.float32)
  a = a + _dup(p["b2"][...])
  return jnp.maximum(_bn(a, p["g2"][...], p["be2"][...]), 0.0)


def _premul_body(x_ref, w_ref, o_ref):
  w = w_ref[...]
  z = jnp.zeros((D, H), jnp.float32)
  wb = jnp.concatenate(
      [jnp.concatenate([w, z], axis=1), jnp.concatenate([z, w], axis=1)],
      axis=0)
  o_ref[...] = jnp.dot(x_ref[...], wb, preferred_element_type=jnp.float32)


def _psum(p_ref):
  return p_ref[pl.ds(0, N2), :] + p_ref[pl.ds(PAD2, N2), :]


def _gin1_body(y_ref, p_ref, pr, o_ref):
  a = y_ref[...] + _psum(p_ref) + _dup(pr["b1"][...])
  a = jnp.maximum(_bn(a, pr["g1"][...], pr["be1"][...]), 0.0)
  o_ref[...] = _mlp_tail(a, pr)


def _gin_body(x_ref, p_ref, pr, o_ref):
  x = x_ref[...]
  o_ref[...] = x + _mlp(x + _psum(p_ref), pr)


def _head(g, p):
  t = jnp.maximum(
      jnp.dot(g, p["W1"][...], preferred_element_type=jnp.float32) + p["b1"][...], 0.0)
  z = jnp.dot(t, p["W2"][...], preferred_element_type=jnp.float32) + p["b2"][...]
  n = jnp.sqrt(jnp.sum(z * z, axis=1, keepdims=True))
  return z / jnp.maximum(n, 1e-12)


def _final_body(x_ref, p_ref, conv3_refs, batch_ref, hc_refs, hs_refs, hg_refs,
                ht_refs, oc_ref, os_ref, og_ref, ot_ref):
  x = x_ref[...]
  h3 = x + _mlp(x + _psum(p_ref), conv3_refs)
  gid = lax.broadcasted_iota(jnp.int32, (N2, G), 1)
  b2 = batch_ref[...]
  oh_e = (b2[:, 0:1] == gid).astype(jnp.float32)
  oh_o = (b2[:, 1:2] == gid).astype(jnp.float32)
  dn = (((0,), (0,)), ((), ()))
  g = (lax.dot_general(oh_e, h3[:, :H], dn, preferred_element_type=jnp.float32)
       + lax.dot_general(oh_o, h3[:, H:], dn,
                         preferred_element_type=jnp.float32))
  oc_ref[...] = _head(g, hc_refs)
  os_ref[...] = _head(g, hs_refs)
  og_ref[...] = _head(g, hg_refs)
  ot_ref[...] = _head(g, ht_refs)


def _nh(shape):
  return jax.ShapeDtypeStruct(shape, jnp.float32)


def pair2(p):
  return p.reshape(2 * PAD2, 2 * H)


def kernel(x, edge_index, batch, params):
  ei = edge_index.reshape(2, NUM_CHUNKS, CHUNK)
  zeros64 = jnp.zeros((SUB_ROWS, H), jnp.float32)

  def pair(a):
    return a.reshape(N2, 2 * H)

  def unpair(a):
    return a.reshape(N, H)

  y = pl.pallas_call(_premul_body, out_shape=_nh((N2, 2 * H)))(
      x.reshape(N2, 2 * D), params["conv1"]["W1"])
  p1 = pair2(_seg_sum_64(unpair(y), ei, zeros64))
  h1 = pl.pallas_call(_gin1_body, out_shape=_nh((N2, 2 * H)))(
      y, p1, params["conv1"])
  p2 = pair2(_seg_sum_64(unpair(h1), ei, zeros64))
  h2 = pl.pallas_call(_gin_body, out_shape=_nh((N2, 2 * H)))(
      h1, p2, params["conv2"])
  p3 = pair2(_seg_sum_64(unpair(h2), ei, zeros64))

  outs = pl.pallas_call(
      _final_body,
      out_shape=(_nh((G, 16)), _nh((G, 16)), _nh((G, 8)), _nh((G, 32))),
  )(h2, p3, params["conv3"], batch.reshape(N2, 2), params["head_color"],
    params["head_size"], params["head_ground"], params["head_struct"])
  return outs
